# baseline (speedup 1.0000x reference)
import functools

import jax
import jax.numpy as jnp
from jax import lax
from jax.experimental import pallas as pl
from jax.experimental.pallas import tpu as pltpu
from jax.experimental.pallas import tpu_sc as plsc

NC = 2
NS = 16
NW = NC * NS
LANES = 16

K = 128
SB = 10
RCH = 80
TPAD = 5008


def _take16(vec, idx):
    dnums = lax.GatherDimensionNumbers(
        offset_dims=(), collapsed_slice_dims=(0,), start_index_map=(0,))
    return lax.gather(vec, idx[:, None], dimension_numbers=dnums,
                      slice_sizes=(1,),
                      mode=lax.GatherScatterMode.PROMISE_IN_BOUNDS)


def _matmul_body(x_ref, w_ref, o_ref):
    o_ref[...] = jnp.dot(x_ref[...], w_ref[0],
                         preferred_element_type=jnp.float32)


def _combine_body(y_ref, p0_ref, p1_ref, b_ref, o_ref):
    o_ref[...] = y_ref[...] + p0_ref[0] + p1_ref[0] + b_ref[...]


def _make_sc_kernel(N, E, D, R):
    NR = N * R
    NBLK = E // K
    NSUP = NBLK // SB
    SE = SB * K
    SL = NR // NS
    NCHUNK = N // RCH
    assert E % (K * SB) == 0 and NR % NS == 0 and N % RCH == 0
    assert SL % 8 == 0 and D % LANES == 0
    assert ((SL + LANES - 1) // LANES) * LANES <= TPAD

    mesh = plsc.VectorSubcoreMesh(core_axis_name="c", subcore_axis_name="s",
                                  num_cores=NC, num_subcores=NS)

    @functools.partial(
        pl.kernel,
        out_type=(jax.ShapeDtypeStruct((NC, N, D), jnp.float32),
                  jax.ShapeDtypeStruct((NR,), jnp.float32)),
        mesh=mesh,
        scratch_types=[
            pltpu.VMEM((SE,), jnp.int32),
            pltpu.VMEM((SB, K), jnp.int32),
            pltpu.VMEM((K,), jnp.float32),
            pltpu.VMEM((TPAD,), jnp.float32),
            pltpu.VMEM((SE,), jnp.int32),
            pltpu.VMEM((SE,), jnp.int32),
            pltpu.VMEM((K,), jnp.int32),
            pltpu.VMEM((K,), jnp.int32),
            pltpu.VMEM((K,), jnp.float32),
            pltpu.VMEM((K,), jnp.float32),
            pltpu.VMEM((K, D), jnp.float32),
            pltpu.VMEM((K, D), jnp.float32),
            pltpu.VMEM_SHARED((NR,), jnp.float32),
            pltpu.VMEM_SHARED((N, D), jnp.float32),
            pltpu.SemaphoreType.DMA,
            pltpu.SemaphoreType.DMA,
        ],
        compiler_params=pltpu.CompilerParams(needs_layout_passes=False),
    )
    def sc_agg(gidx_hbm, key_hbm, dst_hbm, y_hbm, out_hbm, inv_hbm,
               keyc1, keyc2, ones, tbuf, kbuf2, gbuf2, dbA, dbB, wbA, wbB,
               rowsA, rowsB, cnt_sh, acc_sh, semA, semB):
        c = lax.axis_index("c")
        s = lax.axis_index("s")
        wid = c * NS + s

        ones16 = jnp.ones((LANES,), jnp.float32)
        zeros16 = jnp.zeros((LANES,), jnp.float32)

        for g in range(K // LANES):
            ones[pl.ds(g * LANES, LANES)] = ones16

        @plsc.parallel_loop(0, TPAD // LANES, step=1, unroll=4)
        def _(i):
            tbuf[pl.ds(i * LANES, LANES)] = zeros16

        @plsc.parallel_loop(0, K, step=1, unroll=4)
        def _(e):
            for q in range(D // LANES):
                rowsA[e, pl.ds(q * LANES, LANES)] = zeros16

        pltpu.sync_copy(tbuf.at[pl.ds(0, SL)], cnt_sh.at[pl.ds(s * SL, SL)])

        def zero_acc(j, _):
            ck = s + j * NS

            @pl.when(ck < NCHUNK)
            def _():
                cko = pl.multiple_of(ck * RCH, 8)
                pltpu.sync_copy(rowsA.at[pl.ds(0, RCH)],
                                acc_sh.at[pl.ds(cko, RCH)])
            return 0
        lax.fori_loop(0, (NCHUNK + NS - 1) // NS, zero_acc, 0)
        plsc.subcore_barrier()

        def count_sup(j, _):
            sup = s + j * NS

            @pl.when(sup < NSUP)
            def _():
                soff = pl.multiple_of(sup * SE, 8)
                pltpu.sync_copy(key_hbm.at[pl.ds(soff, SE)], keyc1)

                @plsc.parallel_loop(0, SE // LANES, step=1, unroll=4)
                def _(i):
                    r = i // (K // LANES)
                    col = (i % (K // LANES)) * LANES
                    keyc2[r, pl.ds(col, LANES)] = keyc1[pl.ds(i * LANES,
                                                              LANES)]
                cps = [pltpu.async_copy(ones, cnt_sh.at[keyc2.at[q]],
                                        semA, add=True)
                       for q in range(SB)]
                for cp in cps:
                    cp.wait()
            return 0
        lax.fori_loop(0, (NSUP + NS - 1) // NS, count_sup, 0)
        plsc.subcore_barrier()

        pltpu.sync_copy(cnt_sh.at[pl.ds(s * SL, SL)], tbuf.at[pl.ds(0, SL)])

        @plsc.parallel_loop(0, (SL + LANES - 1) // LANES, step=1, unroll=4)
        def _(i):
            v = tbuf[pl.ds(i * LANES, LANES)]
            tbuf[pl.ds(i * LANES, LANES)] = 1.0 / jnp.maximum(v, 1.0)
        pltpu.sync_copy(tbuf.at[pl.ds(0, SL)], inv_hbm.at[pl.ds(s * SL, SL)])
        plsc.subcore_barrier()

        def agg_sup(j, _):
            sup = wid + j * NW

            @pl.when(sup < NSUP)
            def _():
                soff = pl.multiple_of(sup * SE, 8)
                cpg = pltpu.async_copy(gidx_hbm.at[pl.ds(soff, SE)], gbuf2,
                                       semA)
                cpk = pltpu.async_copy(key_hbm.at[pl.ds(soff, SE)], kbuf2,
                                       semB)
                cpg.wait()
                cpk.wait()

                slots = ((rowsA, wbA, dbA, semA), (rowsB, wbB, dbB, semB))

                def issue(b, slot):
                    rbuf, wbuf, dbuf, sem = slot
                    bo = b * K
                    return (
                        pltpu.async_copy(
                            y_hbm.at[gbuf2.at[pl.ds(bo, K)]], rbuf, sem),
                        pltpu.async_copy(
                            inv_hbm.at[kbuf2.at[pl.ds(bo, K)]], wbuf, sem),
                        pltpu.async_copy(
                            dst_hbm.at[pl.ds(soff + bo, K)], dbuf, sem),
                    )

                pend = issue(0, slots[0])
                scat = [None, None]
                for b in range(SB):
                    rbuf, wbuf, dbuf, sem = slots[b % 2]
                    cps = pend
                    if b + 1 < SB:
                        ns = (b + 1) % 2
                        if scat[ns] is not None:
                            scat[ns].wait()
                            scat[ns] = None
                        pend = issue(b + 1, slots[ns])
                    for cp in cps:
                        cp.wait()

                    @plsc.parallel_loop(0, K, step=1, unroll=16)
                    def _(e, rbuf=rbuf, wbuf=wbuf):
                        splat = plsc.load_gather(
                            wbuf, [jnp.broadcast_to(e, (LANES,))])
                        for q in range(D // LANES):
                            sl = pl.ds(q * LANES, LANES)
                            rbuf[e, sl] = rbuf[e, sl] * splat

                    scat[b % 2] = pltpu.async_copy(
                        rbuf, acc_sh.at[dbuf], sem, add=True)
                for pending in scat:
                    if pending is not None:
                        pending.wait()
            return 0
        lax.fori_loop(0, (NSUP + NW - 1) // NW, agg_sup, 0)
        plsc.subcore_barrier()

        def write_out(j, _):
            ck = s + j * NS

            @pl.when(ck < NCHUNK)
            def _():
                cko = pl.multiple_of(ck * RCH, 8)
                pltpu.sync_copy(acc_sh.at[pl.ds(cko, RCH)],
                                rowsA.at[pl.ds(0, RCH)])
                pltpu.sync_copy(rowsA.at[pl.ds(0, RCH)],
                                out_hbm.at[c, pl.ds(cko, RCH)])
            return 0
        lax.fori_loop(0, (NCHUNK + NS - 1) // NS, write_out, 0)

    return sc_agg


def kernel(x, W, root, bias, edge_index, edge_type):
    N, D = x.shape
    R = W.shape[0]
    E = edge_index.shape[1]

    src = edge_index[0]
    dst = edge_index[1]
    gidx = edge_type * N + src
    key = dst * R + edge_type

    Wcat = jnp.concatenate([W, root[None]], axis=0)
    BN = 1000
    NB = N // BN
    ycat = pl.pallas_call(
        _matmul_body,
        grid=(R + 1, NB),
        in_specs=[
            pl.BlockSpec((BN, D), lambda r, i: (i, 0)),
            pl.BlockSpec((1, D, D), lambda r, i: (r, 0, 0)),
        ],
        out_specs=pl.BlockSpec((BN, D), lambda r, i: (r * NB + i, 0)),
        out_shape=jax.ShapeDtypeStruct(((R + 1) * N, D), jnp.float32),
    )(x, Wcat)

    sc_agg = _make_sc_kernel(N, E, D, R)
    partials, _ = sc_agg(gidx, key, dst, ycat)

    out = pl.pallas_call(
        _combine_body,
        grid=(NB,),
        in_specs=[
            pl.BlockSpec((BN, D), lambda i: (R * NB + i, 0)),
            pl.BlockSpec((1, BN, D), lambda i: (0, i, 0)),
            pl.BlockSpec((1, BN, D), lambda i: (1, i, 0)),
            pl.BlockSpec((1, D), lambda i: (0, 0)),
        ],
        out_specs=pl.BlockSpec((BN, D), lambda i: (i, 0)),
        out_shape=jax.ShapeDtypeStruct((N, D), jnp.float32),
    )(ycat, partials, partials, bias[None, :])

    return (out, edge_index, edge_type)


# speedup vs baseline: 1.0158x; 1.0158x over previous
import functools

import jax
import jax.numpy as jnp
from jax import lax
from jax.experimental import pallas as pl
from jax.experimental.pallas import tpu as pltpu
from jax.experimental.pallas import tpu_sc as plsc

NC = 2
NS = 16
NW = NC * NS
LANES = 16

K = 128
SB = 10
RCH = 80
TPAD = 5008


def _take16(vec, idx):
    dnums = lax.GatherDimensionNumbers(
        offset_dims=(), collapsed_slice_dims=(0,), start_index_map=(0,))
    return lax.gather(vec, idx[:, None], dimension_numbers=dnums,
                      slice_sizes=(1,),
                      mode=lax.GatherScatterMode.PROMISE_IN_BOUNDS)


def _matmul_body(x_ref, w_ref, o_ref):
    o_ref[...] = jnp.dot(x_ref[...], w_ref[0],
                         preferred_element_type=jnp.float32)


def _combine_body(y_ref, p0_ref, p1_ref, b_ref, o_ref):
    o_ref[...] = y_ref[...] + p0_ref[0] + p1_ref[0] + b_ref[...]


def _make_sc_kernel(N, E, D, R):
    NR = N * R
    NBLK = E // K
    NSUP = NBLK // SB
    SE = SB * K
    SL = NR // NS
    NCHUNK = N // RCH
    assert E % (K * SB) == 0 and NR % NS == 0 and N % RCH == 0
    assert SL % 8 == 0 and D % LANES == 0
    assert ((SL + LANES - 1) // LANES) * LANES <= TPAD

    mesh = plsc.VectorSubcoreMesh(core_axis_name="c", subcore_axis_name="s",
                                  num_cores=NC, num_subcores=NS)

    @functools.partial(
        pl.kernel,
        out_type=(jax.ShapeDtypeStruct((NC, N, D), jnp.float32),
                  jax.ShapeDtypeStruct((NR,), jnp.float32)),
        mesh=mesh,
        scratch_types=[
            pltpu.VMEM((SE,), jnp.int32),
            pltpu.VMEM((SB, K), jnp.int32),
            pltpu.VMEM((K,), jnp.float32),
            pltpu.VMEM((TPAD,), jnp.float32),
            pltpu.VMEM((SE,), jnp.int32),
            pltpu.VMEM((SE,), jnp.int32),
            pltpu.VMEM((K,), jnp.int32),
            pltpu.VMEM((K,), jnp.int32),
            pltpu.VMEM((K,), jnp.float32),
            pltpu.VMEM((K,), jnp.float32),
            pltpu.VMEM((K, D), jnp.float32),
            pltpu.VMEM((K, D), jnp.float32),
            pltpu.VMEM_SHARED((NR,), jnp.float32),
            pltpu.VMEM_SHARED((N, D), jnp.float32),
            pltpu.SemaphoreType.DMA,
            pltpu.SemaphoreType.DMA,
        ],
        compiler_params=pltpu.CompilerParams(needs_layout_passes=False),
    )
    def sc_agg(gidx_hbm, key_hbm, dst_hbm, y_hbm, out_hbm, inv_hbm,
               keyc1, keyc2, ones, tbuf, kbuf2, gbuf2, dbA, dbB, wbA, wbB,
               rowsA, rowsB, cnt_sh, acc_sh, semA, semB):
        c = lax.axis_index("c")
        s = lax.axis_index("s")
        wid = c * NS + s

        ones16 = jnp.ones((LANES,), jnp.float32)
        zeros16 = jnp.zeros((LANES,), jnp.float32)

        for g in range(K // LANES):
            ones[pl.ds(g * LANES, LANES)] = ones16

        @plsc.parallel_loop(0, TPAD // LANES, step=1, unroll=4)
        def _(i):
            tbuf[pl.ds(i * LANES, LANES)] = zeros16

        @plsc.parallel_loop(0, K, step=1, unroll=4)
        def _(e):
            for q in range(D // LANES):
                rowsA[e, pl.ds(q * LANES, LANES)] = zeros16

        pltpu.sync_copy(tbuf.at[pl.ds(0, SL)], cnt_sh.at[pl.ds(s * SL, SL)])

        def zero_acc(j, _):
            ck = s + j * NS

            @pl.when(ck < NCHUNK)
            def _():
                cko = pl.multiple_of(ck * RCH, 8)
                pltpu.sync_copy(rowsA.at[pl.ds(0, RCH)],
                                acc_sh.at[pl.ds(cko, RCH)])
            return 0
        lax.fori_loop(0, (NCHUNK + NS - 1) // NS, zero_acc, 0)
        plsc.subcore_barrier()

        def count_sup(j, _):
            sup = s + j * NS

            @pl.when(sup < NSUP)
            def _():
                soff = pl.multiple_of(sup * SE, 8)
                pltpu.sync_copy(key_hbm.at[pl.ds(soff, SE)], keyc1)

                @plsc.parallel_loop(0, SE // LANES, step=1, unroll=4)
                def _(i):
                    r = i // (K // LANES)
                    col = (i % (K // LANES)) * LANES
                    keyc2[r, pl.ds(col, LANES)] = keyc1[pl.ds(i * LANES,
                                                              LANES)]
                cps = [pltpu.async_copy(ones, cnt_sh.at[keyc2.at[q]],
                                        semA, add=True)
                       for q in range(SB)]
                for cp in cps:
                    cp.wait()
            return 0
        lax.fori_loop(0, (NSUP + NS - 1) // NS, count_sup, 0)
        plsc.subcore_barrier()

        pltpu.sync_copy(cnt_sh.at[pl.ds(s * SL, SL)], tbuf.at[pl.ds(0, SL)])

        @plsc.parallel_loop(0, (SL + LANES - 1) // LANES, step=1, unroll=4)
        def _(i):
            v = tbuf[pl.ds(i * LANES, LANES)]
            tbuf[pl.ds(i * LANES, LANES)] = 1.0 / jnp.maximum(v, 1.0)
        pltpu.sync_copy(tbuf.at[pl.ds(0, SL)], inv_hbm.at[pl.ds(s * SL, SL)])
        plsc.subcore_barrier()

        def agg_sup(j, _):
            sup = wid + j * NW

            @pl.when(sup < NSUP)
            def _():
                soff = pl.multiple_of(sup * SE, 8)
                cpg = pltpu.async_copy(gidx_hbm.at[pl.ds(soff, SE)], gbuf2,
                                       semA)
                cpk = pltpu.async_copy(key_hbm.at[pl.ds(soff, SE)], kbuf2,
                                       semB)
                cpg.wait()
                cpk.wait()

                slots = ((rowsA, wbA, dbA, semA), (rowsB, wbB, dbB, semB))

                def issue(b, slot):
                    rbuf, wbuf, dbuf, sem = slot
                    bo = b * K
                    return (
                        pltpu.async_copy(
                            y_hbm.at[gbuf2.at[pl.ds(bo, K)]], rbuf, sem),
                        pltpu.async_copy(
                            inv_hbm.at[kbuf2.at[pl.ds(bo, K)]], wbuf, sem),
                        pltpu.async_copy(
                            dst_hbm.at[pl.ds(soff + bo, K)], dbuf, sem),
                    )

                pend = issue(0, slots[0])
                scat = [None, None]
                for b in range(SB):
                    rbuf, wbuf, dbuf, sem = slots[b % 2]
                    cps = pend
                    if b + 1 < SB:
                        ns = (b + 1) % 2
                        if scat[ns] is not None:
                            scat[ns].wait()
                            scat[ns] = None
                        pend = issue(b + 1, slots[ns])
                    for cp in cps:
                        cp.wait()

                    @plsc.parallel_loop(0, K, step=1, unroll=8)
                    def _(e, rbuf=rbuf, wbuf=wbuf):
                        splat = plsc.load_gather(
                            wbuf, [jnp.broadcast_to(e, (LANES,))])
                        for q in range(D // LANES):
                            sl = pl.ds(q * LANES, LANES)
                            rbuf[e, sl] = rbuf[e, sl] * splat

                    scat[b % 2] = pltpu.async_copy(
                        rbuf, acc_sh.at[dbuf], sem, add=True)
                for pending in scat:
                    if pending is not None:
                        pending.wait()
            return 0
        lax.fori_loop(0, (NSUP + NW - 1) // NW, agg_sup, 0)
        plsc.subcore_barrier()

        def write_out(j, _):
            ck = s + j * NS

            @pl.when(ck < NCHUNK)
            def _():
                cko = pl.multiple_of(ck * RCH, 8)
                pltpu.sync_copy(acc_sh.at[pl.ds(cko, RCH)],
                                rowsA.at[pl.ds(0, RCH)])
                pltpu.sync_copy(rowsA.at[pl.ds(0, RCH)],
                                out_hbm.at[c, pl.ds(cko, RCH)])
            return 0
        lax.fori_loop(0, (NCHUNK + NS - 1) // NS, write_out, 0)

    return sc_agg


def kernel(x, W, root, bias, edge_index, edge_type):
    N, D = x.shape
    R = W.shape[0]
    E = edge_index.shape[1]

    src = edge_index[0]
    dst = edge_index[1]
    gidx = edge_type * N + src
    key = dst * R + edge_type

    Wcat = jnp.concatenate([W, root[None]], axis=0)
    BN = 1000
    NB = N // BN
    ycat = pl.pallas_call(
        _matmul_body,
        grid=(R + 1, NB),
        in_specs=[
            pl.BlockSpec((BN, D), lambda r, i: (i, 0)),
            pl.BlockSpec((1, D, D), lambda r, i: (r, 0, 0)),
        ],
        out_specs=pl.BlockSpec((BN, D), lambda r, i: (r * NB + i, 0)),
        out_shape=jax.ShapeDtypeStruct(((R + 1) * N, D), jnp.float32),
    )(x, Wcat)

    sc_agg = _make_sc_kernel(N, E, D, R)
    partials, _ = sc_agg(gidx, key, dst, ycat)

    out = pl.pallas_call(
        _combine_body,
        grid=(NB,),
        in_specs=[
            pl.BlockSpec((BN, D), lambda i: (R * NB + i, 0)),
            pl.BlockSpec((1, BN, D), lambda i: (0, i, 0)),
            pl.BlockSpec((1, BN, D), lambda i: (1, i, 0)),
            pl.BlockSpec((1, D), lambda i: (0, 0)),
        ],
        out_specs=pl.BlockSpec((BN, D), lambda i: (i, 0)),
        out_shape=jax.ShapeDtypeStruct((N, D), jnp.float32),
    )(ycat, partials, partials, bias[None, :])

    return (out, edge_index, edge_type)
